# baseline (device time: 20060 ns/iter reference)
import jax
import jax.numpy as jnp
from jax import lax
from jax.experimental import pallas as pl
from jax.experimental.pallas import tpu as pltpu


def kernel(x):
    m, n = x.shape
    h = m // 2
    out_dtype = jnp.bfloat16

    def body(x_ref, out_ref, send1, recv1, send2, recv2):
        my_x = lax.axis_index("x")
        my_y = lax.axis_index("y")
        my_z = lax.axis_index("z")
        ynbr = (my_x, 1 - my_y, my_z)
        xnbr = (1 - my_x, my_y, my_z)

        barrier_sem = pltpu.get_barrier_semaphore()
        for nbr in (ynbr, xnbr):
            pl.semaphore_signal(
                barrier_sem, inc=1, device_id=nbr,
                device_id_type=pl.DeviceIdType.MESH,
            )
        pl.semaphore_wait(barrier_sem, 2)

        own = my_y * m

        send_off = own + my_x * h
        out_ref[pl.ds(send_off, h), :] = x_ref[pl.ds(my_x * h, h), :].astype(
            out_dtype
        )
        rdma1 = pltpu.make_async_remote_copy(
            src_ref=out_ref.at[pl.ds(send_off, h), :],
            dst_ref=out_ref.at[pl.ds(send_off, h), :],
            send_sem=send1,
            recv_sem=recv1,
            device_id=ynbr,
            device_id_type=pl.DeviceIdType.MESH,
        )
        rdma1.start()

        keep_off = own + (1 - my_x) * h
        out_ref[pl.ds(keep_off, h), :] = x_ref[
            pl.ds((1 - my_x) * h, h), :
        ].astype(out_dtype)

        rdma1.wait_recv()

        fwd_off = (1 - my_y) * m + my_x * h
        rdma2 = pltpu.make_async_remote_copy(
            src_ref=out_ref.at[pl.ds(fwd_off, h), :],
            dst_ref=out_ref.at[pl.ds(fwd_off, h), :],
            send_sem=send2,
            recv_sem=recv2,
            device_id=xnbr,
            device_id_type=pl.DeviceIdType.MESH,
        )
        rdma2.start()

        rdma1.wait_send()
        rdma2.wait()

    return pl.pallas_call(
        body,
        out_shape=jax.ShapeDtypeStruct((2 * m, n), out_dtype),
        in_specs=[pl.BlockSpec(memory_space=pltpu.VMEM)],
        out_specs=pl.BlockSpec(memory_space=pltpu.VMEM),
        scratch_shapes=[
            pltpu.SemaphoreType.DMA,
            pltpu.SemaphoreType.DMA,
            pltpu.SemaphoreType.DMA,
            pltpu.SemaphoreType.DMA,
        ],
        compiler_params=pltpu.CompilerParams(collective_id=0),
    )(x)


# device time: 15947 ns/iter; 1.2579x vs baseline; 1.2579x over previous
import jax
import jax.numpy as jnp
from jax import lax
from jax.experimental import pallas as pl
from jax.experimental.pallas import tpu as pltpu

NBLK = 4


def kernel(x):
    m, n = x.shape
    h = m // 2
    blk = h // NBLK
    out_dtype = jnp.bfloat16

    def body(x_ref, out_ref, send1, recv1, send2, recv2):
        my_x = lax.axis_index("x")
        my_y = lax.axis_index("y")
        my_z = lax.axis_index("z")
        ynbr = (my_x, 1 - my_y, my_z)
        xnbr = (1 - my_x, my_y, my_z)

        barrier_sem = pltpu.get_barrier_semaphore()
        for nbr in (ynbr, xnbr):
            pl.semaphore_signal(
                barrier_sem, inc=1, device_id=nbr,
                device_id_type=pl.DeviceIdType.MESH,
            )
        pl.semaphore_wait(barrier_sem, 2)

        own = my_y * m
        send_off = own + my_x * h
        fwd_off = (1 - my_y) * m + my_x * h

        def p1(b):
            return pltpu.make_async_remote_copy(
                src_ref=out_ref.at[pl.ds(send_off + b * blk, blk), :],
                dst_ref=out_ref.at[pl.ds(send_off + b * blk, blk), :],
                send_sem=send1.at[b],
                recv_sem=recv1.at[b],
                device_id=ynbr,
                device_id_type=pl.DeviceIdType.MESH,
            )

        def p2(b):
            return pltpu.make_async_remote_copy(
                src_ref=out_ref.at[pl.ds(fwd_off + b * blk, blk), :],
                dst_ref=out_ref.at[pl.ds(fwd_off + b * blk, blk), :],
                send_sem=send2.at[b],
                recv_sem=recv2.at[b],
                device_id=xnbr,
                device_id_type=pl.DeviceIdType.MESH,
            )

        for b in range(NBLK):
            r0 = send_off + b * blk
            out_ref[pl.ds(r0, blk), :] = x_ref[
                pl.ds(my_x * h + b * blk, blk), :
            ].astype(out_dtype)
            p1(b).start()

        keep_off = own + (1 - my_x) * h
        out_ref[pl.ds(keep_off, h), :] = x_ref[
            pl.ds((1 - my_x) * h, h), :
        ].astype(out_dtype)

        for b in range(NBLK):
            p1(b).wait_recv()
            p2(b).start()

        for b in range(NBLK):
            p1(b).wait_send()
            p2(b).wait_send()
        for b in range(NBLK):
            p2(b).wait_recv()

    return pl.pallas_call(
        body,
        out_shape=jax.ShapeDtypeStruct((2 * m, n), out_dtype),
        in_specs=[pl.BlockSpec(memory_space=pltpu.VMEM)],
        out_specs=pl.BlockSpec(memory_space=pltpu.VMEM),
        scratch_shapes=[
            pltpu.SemaphoreType.DMA((NBLK,)),
            pltpu.SemaphoreType.DMA((NBLK,)),
            pltpu.SemaphoreType.DMA((NBLK,)),
            pltpu.SemaphoreType.DMA((NBLK,)),
        ],
        compiler_params=pltpu.CompilerParams(collective_id=0),
    )(x)


# device time: 15334 ns/iter; 1.3082x vs baseline; 1.0400x over previous
import jax
import jax.numpy as jnp
from jax import lax
from jax.experimental import pallas as pl
from jax.experimental.pallas import tpu as pltpu

NBLK = 8


def kernel(x):
    m, n = x.shape
    h = m // 2
    blk = h // NBLK
    out_dtype = jnp.bfloat16

    def body(x_ref, out_ref, send1, recv1, send2, recv2):
        my_x = lax.axis_index("x")
        my_y = lax.axis_index("y")
        my_z = lax.axis_index("z")
        ynbr = (my_x, 1 - my_y, my_z)
        xnbr = (1 - my_x, my_y, my_z)

        barrier_sem = pltpu.get_barrier_semaphore()
        for nbr in (ynbr, xnbr):
            pl.semaphore_signal(
                barrier_sem, inc=1, device_id=nbr,
                device_id_type=pl.DeviceIdType.MESH,
            )
        pl.semaphore_wait(barrier_sem, 2)

        own = my_y * m
        send_off = own + my_x * h
        fwd_off = (1 - my_y) * m + my_x * h

        def p1(b):
            return pltpu.make_async_remote_copy(
                src_ref=out_ref.at[pl.ds(send_off + b * blk, blk), :],
                dst_ref=out_ref.at[pl.ds(send_off + b * blk, blk), :],
                send_sem=send1.at[b],
                recv_sem=recv1.at[b],
                device_id=ynbr,
                device_id_type=pl.DeviceIdType.MESH,
            )

        def p2(b):
            return pltpu.make_async_remote_copy(
                src_ref=out_ref.at[pl.ds(fwd_off + b * blk, blk), :],
                dst_ref=out_ref.at[pl.ds(fwd_off + b * blk, blk), :],
                send_sem=send2.at[b],
                recv_sem=recv2.at[b],
                device_id=xnbr,
                device_id_type=pl.DeviceIdType.MESH,
            )

        for b in range(NBLK):
            r0 = send_off + b * blk
            out_ref[pl.ds(r0, blk), :] = x_ref[
                pl.ds(my_x * h + b * blk, blk), :
            ].astype(out_dtype)
            p1(b).start()

        keep_off = own + (1 - my_x) * h
        out_ref[pl.ds(keep_off, h), :] = x_ref[
            pl.ds((1 - my_x) * h, h), :
        ].astype(out_dtype)

        for b in range(NBLK):
            p1(b).wait_recv()
            p2(b).start()

        for b in range(NBLK):
            p1(b).wait_send()
            p2(b).wait_send()
        for b in range(NBLK):
            p2(b).wait_recv()

    return pl.pallas_call(
        body,
        out_shape=jax.ShapeDtypeStruct((2 * m, n), out_dtype),
        in_specs=[pl.BlockSpec(memory_space=pltpu.VMEM)],
        out_specs=pl.BlockSpec(memory_space=pltpu.VMEM),
        scratch_shapes=[
            pltpu.SemaphoreType.DMA((NBLK,)),
            pltpu.SemaphoreType.DMA((NBLK,)),
            pltpu.SemaphoreType.DMA((NBLK,)),
            pltpu.SemaphoreType.DMA((NBLK,)),
        ],
        compiler_params=pltpu.CompilerParams(collective_id=0),
    )(x)
